# baseline (speedup 1.0000x reference)
import functools

import jax
import jax.numpy as jnp
from jax import lax
from jax.experimental import pallas as pl
from jax.experimental.pallas import tpu as pltpu
from jax.experimental.pallas import tpu_sc as plsc

_EPS = 1e-12
_DIM = 64
_W = 512
_N = 1_000_000
_NCHUNK = _N // _W
_TAIL = _N - _NCHUNK * _W
_L = 16


def _rsqrt_nr(s):
    i = lax.bitcast_convert_type(s, jnp.int32)
    y = lax.bitcast_convert_type(
        jnp.int32(0x5F3759DF) - lax.shift_right_arithmetic(i, 1), jnp.float32)
    for _ in range(3):
        y = y * (1.5 - 0.5 * s * y * y)
    return y


def _normalize_chunk(xv, ov, width):
    def group(g, _):
        c0 = g * _L
        acc = jnp.zeros((_L,), jnp.float32)
        for r in range(_DIM):
            v = xv[r, pl.ds(c0, _L)]
            acc = acc + v * v
        rs = _rsqrt_nr(jnp.maximum(acc, jnp.float32(_EPS * _EPS)))
        for r in range(_DIM):
            ov[r, pl.ds(c0, _L)] = xv[r, pl.ds(c0, _L)] * rs
        return _

    lax.fori_loop(0, width // _L, group, 0)


def _sc_body(x_hbm, o_hbm, xv, ov, txv, tov, sem):
    info = plsc.get_sparse_core_info()
    nw = info.num_cores * info.num_subcores
    wid = lax.axis_index("s") * info.num_cores + lax.axis_index("c")
    nloop = (_NCHUNK + nw - 1) // nw

    def step(k, _):
        g = k * nw + wid

        @pl.when(g < _NCHUNK)
        def _do():
            base = g * _W
            pltpu.async_copy(x_hbm.at[:, pl.ds(base, _W)], xv, sem).wait()
            _normalize_chunk(xv, ov, _W)
            pltpu.async_copy(ov, o_hbm.at[:, pl.ds(base, _W)], sem).wait()

        return _

    lax.fori_loop(0, nloop, step, 0)

    @pl.when(wid == 0)
    def _tail():
        base = _NCHUNK * _W
        pltpu.async_copy(x_hbm.at[:, pl.ds(base, _TAIL)], txv, sem).wait()
        _normalize_chunk(txv, tov, _TAIL)
        pltpu.async_copy(tov, o_hbm.at[:, pl.ds(base, _TAIL)], sem).wait()


def kernel(weight):
    n_rows, dim = weight.shape
    wt = weight.T
    mesh = plsc.VectorSubcoreMesh(core_axis_name="c", subcore_axis_name="s")
    sc = functools.partial(
        pl.kernel,
        out_type=jax.ShapeDtypeStruct((dim, n_rows), weight.dtype),
        mesh=mesh,
        scratch_types=[
            pltpu.VMEM((_DIM, _W), jnp.float32),
            pltpu.VMEM((_DIM, _W), jnp.float32),
            pltpu.VMEM((_DIM, _TAIL), jnp.float32),
            pltpu.VMEM((_DIM, _TAIL), jnp.float32),
            pltpu.SemaphoreType.DMA,
        ],
    )(_sc_body)
    return sc(wt).T


# speedup vs baseline: 4.6956x; 4.6956x over previous
import jax
import jax.numpy as jnp
from jax.experimental import pallas as pl
from jax.experimental.pallas import tpu as pltpu

_EPS = 1e-12
_K = 12
_CH = 3968
_DIM = 64
_TAIL = 64


def _normalize(x):
    s = jnp.sum(x * x, axis=0, keepdims=True)
    return x * jax.lax.rsqrt(jnp.maximum(s, _EPS * _EPS))


def _body(x_hbm, o_hbm, in_buf, out_buf, tin, tout, in_sem, out_sem, tsem):
    step = pl.program_id(0)
    nsteps = pl.num_programs(0)
    cols_per_step = _K * _CH

    def in_copy(slot, s):
        base = s * cols_per_step + slot * _CH
        return pltpu.make_async_copy(
            x_hbm.at[:, pl.ds(base, _CH)], in_buf.at[slot], in_sem.at[slot])

    def out_copy(slot, s):
        base = s * cols_per_step + slot * _CH
        return pltpu.make_async_copy(
            out_buf.at[slot], o_hbm.at[:, pl.ds(base, _CH)], out_sem.at[slot])

    @pl.when(step == 0)
    def _prologue():
        for j in range(_K):
            in_copy(j, 0).start()

    for j in range(_K):
        in_copy(j, step).wait()

        @pl.when(step > 0)
        def _slot_free():
            out_copy(j, step - 1).wait()

        out_buf[j] = _normalize(in_buf[j])

        @pl.when(step + 1 < nsteps)
        def _prefetch():
            in_copy(j, step + 1).start()

        out_copy(j, step).start()

    @pl.when(step == nsteps - 1)
    def _epilogue():
        base = nsteps * cols_per_step
        pltpu.make_async_copy(
            x_hbm.at[:, pl.ds(base, _TAIL)], tin, tsem).start()
        pltpu.make_async_copy(
            x_hbm.at[:, pl.ds(base, _TAIL)], tin, tsem).wait()
        tout[...] = _normalize(tin[...])
        pltpu.make_async_copy(
            tout, o_hbm.at[:, pl.ds(base, _TAIL)], tsem).start()
        pltpu.make_async_copy(
            tout, o_hbm.at[:, pl.ds(base, _TAIL)], tsem).wait()
        for j in range(_K):
            out_copy(j, step).wait()


def kernel(weight):
    n_rows, dim = weight.shape
    wt = weight.T
    nsteps = (n_rows - _TAIL) // (_K * _CH)
    out_t = pl.pallas_call(
        _body,
        grid=(nsteps,),
        in_specs=[pl.BlockSpec(memory_space=pltpu.MemorySpace.HBM)],
        out_specs=pl.BlockSpec(memory_space=pltpu.MemorySpace.HBM),
        out_shape=jax.ShapeDtypeStruct((dim, n_rows), weight.dtype),
        scratch_shapes=[
            pltpu.VMEM((_K, _DIM, _CH), jnp.float32),
            pltpu.VMEM((_K, _DIM, _CH), jnp.float32),
            pltpu.VMEM((_DIM, _TAIL), jnp.float32),
            pltpu.VMEM((_DIM, _TAIL), jnp.float32),
            pltpu.SemaphoreType.DMA((_K,)),
            pltpu.SemaphoreType.DMA((_K,)),
            pltpu.SemaphoreType.DMA,
        ],
    )(wt)
    return out_t.T
